# baseline (speedup 1.0000x reference)
import functools

import jax
import jax.numpy as jnp
from jax import lax
from jax.experimental import pallas as pl
from jax.experimental.pallas import tpu as pltpu
from jax.experimental.pallas import tpu_sc as plsc

H = 64
B = 16384
NC, NS = 2, 16
NW = NC * NS
BPW = B // NW
CHUNK = 128
NCHUNK = BPW // CHUNK


def _gather_body(idx_hbm, table_hbm, out_hbm, idx_v, rows_v, sem):
  wid = lax.axis_index("s") * NC + lax.axis_index("c")
  base = wid * BPW
  pltpu.sync_copy(idx_hbm.at[wid], idx_v)
  copies = [
      pltpu.async_copy(
          table_hbm.at[idx_v.at[j]],
          rows_v.at[pl.ds(j * CHUNK, CHUNK)],
          sem,
      )
      for j in range(NCHUNK)
  ]
  for c in copies:
    c.wait()
  pltpu.sync_copy(rows_v, out_hbm.at[pl.ds(base, BPW)])


@jax.jit
def _gather(idx, table):
  mesh = plsc.VectorSubcoreMesh(core_axis_name="c", subcore_axis_name="s")
  return pl.kernel(
      _gather_body,
      out_type=jax.ShapeDtypeStruct((B, H), jnp.float32),
      mesh=mesh,
      scratch_types=[
          pltpu.VMEM((NCHUNK, CHUNK), jnp.int32),
          pltpu.VMEM((BPW, H), jnp.float32),
          pltpu.SemaphoreType.DMA,
      ],
      compiler_params=pltpu.CompilerParams(use_tc_tiling_on_sc=False),
  )(idx, table)


RB = 2048


def _mlp_body(f_ref, w1_ref, b1_ref, w2t_ref, b2_ref, o_ref):
  h = jnp.dot(f_ref[...], w1_ref[...], preferred_element_type=jnp.float32)
  h = jnp.maximum(h + b1_ref[...], 0.0)
  o_ref[...] = (
      jnp.sum(h * w2t_ref[...], axis=1, keepdims=True) + b2_ref[...]
  )


@jax.jit
def _mlp(feats, w1, b1r, w2t, b2r):
  return pl.pallas_call(
      _mlp_body,
      grid=(B // RB,),
      in_specs=[
          pl.BlockSpec((RB, H), lambda i: (i, 0)),
          pl.BlockSpec((H, H), lambda i: (0, 0)),
          pl.BlockSpec((1, H), lambda i: (0, 0)),
          pl.BlockSpec((1, H), lambda i: (0, 0)),
          pl.BlockSpec((1, 1), lambda i: (0, 0)),
      ],
      out_specs=pl.BlockSpec((RB, 1), lambda i: (i, 0)),
      out_shape=jax.ShapeDtypeStruct((B, 1), jnp.float32),
  )(feats, w1, b1r, w2t, b2r)


def kernel(state_indices, embedding, W1, b1, W2, b2):
  idx = state_indices.astype(jnp.int32).reshape(NW, NCHUNK, CHUNK)
  feats = _gather(idx, embedding)
  out = _mlp(
      feats,
      W1,
      b1.reshape(1, H),
      W2.reshape(1, H),
      b2.reshape(1, 1),
  )
  return out.reshape(B)
